# speedup vs baseline: 2.4367x; 2.4367x over previous
import functools

import jax
import jax.numpy as jnp
from jax import lax
from jax.experimental import pallas as pl
from jax.experimental.pallas import tpu as pltpu
from jax.experimental.pallas import tpu_sc as plsc

NC = 2
NS = 16
NW = NC * NS

BATCH = 16384
DIM = 128
B_PER_W = BATCH // NW
CHUNK = 128
N_CHUNKS = B_PER_W // CHUNK


def _sc_gather3(head2, rel2, tail2, ent, rtab):
    mesh = plsc.VectorSubcoreMesh(
        core_axis_name="c", subcore_axis_name="s", num_cores=NC, num_subcores=NS
    )
    out_t = (
        jax.ShapeDtypeStruct((BATCH, DIM), jnp.float32),
        jax.ShapeDtypeStruct((BATCH, DIM), jnp.float32),
        jax.ShapeDtypeStruct((BATCH, DIM), jnp.float32),
    )

    @functools.partial(
        pl.kernel,
        out_type=out_t,
        mesh=mesh,
        scratch_types=[
            pltpu.VMEM((N_CHUNKS, CHUNK), jnp.int32),
            pltpu.VMEM((CHUNK, DIM), jnp.float32),
            pltpu.VMEM((CHUNK, DIM), jnp.float32),
            pltpu.SemaphoreType.DMA,
            pltpu.SemaphoreType.DMA,
        ],
    )
    def k(head_h, rel_h, tail_h, ent_h, rtab_h, ho, ro, to, idx_v, rows0, rows1, sem0, sem1):
        wid = lax.axis_index("s") * NC + lax.axis_index("c")
        rbase = wid * N_CHUNKS
        obase = wid * B_PER_W

        rows = (rows0, rows1)
        sems = (sem0, sem1)

        for idx_h, tab_h, out_h in ((head_h, ent_h, ho), (rel_h, rtab_h, ro), (tail_h, ent_h, to)):
            pltpu.sync_copy(idx_h.at[pl.ds(rbase, N_CHUNKS)], idx_v)
            pltpu.async_copy(tab_h.at[idx_v.at[0]], rows[0], sems[0])
            for c in range(N_CHUNKS):
                if c + 1 < N_CHUNKS:
                    pltpu.async_copy(tab_h.at[idx_v.at[c + 1]], rows[(c + 1) % 2], sems[(c + 1) % 2])
                pltpu.make_async_copy(tab_h.at[idx_v.at[c]], rows[c % 2], sems[c % 2]).wait()
                pltpu.sync_copy(rows[c % 2], out_h.at[pl.ds(obase + c * CHUNK, CHUNK)])

    return k(head2, rel2, tail2, ent, rtab)


@jax.jit
def kernel(head, rel, tail, entity_embedding, rel_embedding):
    head2 = head.astype(jnp.int32).reshape(BATCH // CHUNK, CHUNK)
    rel2 = rel.astype(jnp.int32).reshape(BATCH // CHUNK, CHUNK)
    tail2 = tail.astype(jnp.int32).reshape(BATCH // CHUNK, CHUNK)
    return _sc_gather3(head2, rel2, tail2, entity_embedding, rel_embedding)


# speedup vs baseline: 2.6261x; 1.0777x over previous
import functools

import jax
import jax.numpy as jnp
from jax import lax
from jax.experimental import pallas as pl
from jax.experimental.pallas import tpu as pltpu
from jax.experimental.pallas import tpu_sc as plsc

NC = 2
NS = 16
NW = NC * NS

BATCH = 16384
DIM = 128
B_PER_W = BATCH // NW
CHUNK = 128
N_CHUNKS = B_PER_W // CHUNK


def _sc_gather3(head2, rel2, tail2, ent, rtab):
    mesh = plsc.VectorSubcoreMesh(
        core_axis_name="c", subcore_axis_name="s", num_cores=NC, num_subcores=NS
    )
    out_t = (
        jax.ShapeDtypeStruct((BATCH, DIM), jnp.float32),
        jax.ShapeDtypeStruct((BATCH, DIM), jnp.float32),
        jax.ShapeDtypeStruct((BATCH, DIM), jnp.float32),
    )

    NBUF = 6
    NJOBS = 3 * N_CHUNKS

    @functools.partial(
        pl.kernel,
        out_type=out_t,
        mesh=mesh,
        scratch_types=[
            pltpu.VMEM((NJOBS, CHUNK), jnp.int32),
            [pltpu.VMEM((CHUNK, DIM), jnp.float32) for _ in range(NBUF)],
            [pltpu.SemaphoreType.DMA for _ in range(NBUF)],
            [pltpu.SemaphoreType.DMA for _ in range(NBUF)],
            pltpu.SemaphoreType.DMA,
        ],
    )
    def k(head_h, rel_h, tail_h, ent_h, rtab_h, ho, ro, to, idx_v, rows, gsems, osems, isem):
        wid = lax.axis_index("s") * NC + lax.axis_index("c")
        rbase = wid * N_CHUNKS
        obase = wid * B_PER_W

        tabs = (ent_h, rtab_h, ent_h)
        outs = (ho, ro, to)

        for j, idx_h in enumerate((head_h, rel_h, tail_h)):
            pltpu.async_copy(idx_h.at[pl.ds(rbase, N_CHUNKS)],
                             idx_v.at[pl.ds(j * N_CHUNKS, N_CHUNKS)], isem)
        for j, idx_h in enumerate((head_h, rel_h, tail_h)):
            pltpu.make_async_copy(idx_h.at[pl.ds(rbase, N_CHUNKS)],
                                  idx_v.at[pl.ds(j * N_CHUNKS, N_CHUNKS)], isem).wait()

        def gather(c, b):
            pltpu.async_copy(tabs[c // N_CHUNKS].at[idx_v.at[c]], rows[b], gsems[b])

        def out_slice(c):
            return outs[c // N_CHUNKS].at[pl.ds(obase + (c % N_CHUNKS) * CHUNK, CHUNK)]

        for c in range(NBUF):
            gather(c, c)
        for c in range(NJOBS):
            b = c % NBUF
            pltpu.make_async_copy(tabs[c // N_CHUNKS].at[idx_v.at[c]], rows[b], gsems[b]).wait()
            pltpu.async_copy(rows[b], out_slice(c), osems[b])
            nc = c + NBUF
            if nc < NJOBS:
                pltpu.make_async_copy(rows[b], out_slice(c), osems[b]).wait()
                gather(nc, b)
        for c in range(NJOBS - NBUF, NJOBS):
            b = c % NBUF
            pltpu.make_async_copy(rows[b], out_slice(c), osems[b]).wait()

    return k(head2, rel2, tail2, ent, rtab)


@jax.jit
def kernel(head, rel, tail, entity_embedding, rel_embedding):
    head2 = head.astype(jnp.int32).reshape(BATCH // CHUNK, CHUNK)
    rel2 = rel.astype(jnp.int32).reshape(BATCH // CHUNK, CHUNK)
    tail2 = tail.astype(jnp.int32).reshape(BATCH // CHUNK, CHUNK)
    return _sc_gather3(head2, rel2, tail2, entity_embedding, rel_embedding)


# speedup vs baseline: 2.6274x; 1.0005x over previous
import functools

import jax
import jax.numpy as jnp
from jax import lax
from jax.experimental import pallas as pl
from jax.experimental.pallas import tpu as pltpu
from jax.experimental.pallas import tpu_sc as plsc

NC = 2
NS = 16
NW = NC * NS

BATCH = 16384
DIM = 128
B_PER_W = BATCH // NW
CHUNK = 128
N_CHUNKS = B_PER_W // CHUNK


def _sc_gather3(head2, rel2, tail2, ent, rtab):
    mesh = plsc.VectorSubcoreMesh(
        core_axis_name="c", subcore_axis_name="s", num_cores=NC, num_subcores=NS
    )
    out_t = (
        jax.ShapeDtypeStruct((BATCH, DIM), jnp.float32),
        jax.ShapeDtypeStruct((BATCH, DIM), jnp.float32),
        jax.ShapeDtypeStruct((BATCH, DIM), jnp.float32),
    )

    NBUF = 7
    NJOBS = 3 * N_CHUNKS

    @functools.partial(
        pl.kernel,
        out_type=out_t,
        mesh=mesh,
        scratch_types=[
            pltpu.VMEM((NJOBS, CHUNK), jnp.int32),
            [pltpu.VMEM((CHUNK, DIM), jnp.float32) for _ in range(NBUF)],
            [pltpu.SemaphoreType.DMA for _ in range(NBUF)],
            [pltpu.SemaphoreType.DMA for _ in range(NBUF)],
            pltpu.SemaphoreType.DMA,
        ],
    )
    def k(head_h, rel_h, tail_h, ent_h, rtab_h, ho, ro, to, idx_v, rows, gsems, osems, isem):
        wid = lax.axis_index("s") * NC + lax.axis_index("c")
        rbase = wid * N_CHUNKS
        obase = wid * B_PER_W

        tabs = (ent_h, rtab_h, ent_h)
        outs = (ho, ro, to)

        for j, idx_h in enumerate((head_h, rel_h, tail_h)):
            pltpu.async_copy(idx_h.at[pl.ds(rbase, N_CHUNKS)],
                             idx_v.at[pl.ds(j * N_CHUNKS, N_CHUNKS)], isem)
        for j, idx_h in enumerate((head_h, rel_h, tail_h)):
            pltpu.make_async_copy(idx_h.at[pl.ds(rbase, N_CHUNKS)],
                                  idx_v.at[pl.ds(j * N_CHUNKS, N_CHUNKS)], isem).wait()

        def gather(c, b):
            pltpu.async_copy(tabs[c // N_CHUNKS].at[idx_v.at[c]], rows[b], gsems[b])

        def out_slice(c):
            return outs[c // N_CHUNKS].at[pl.ds(obase + (c % N_CHUNKS) * CHUNK, CHUNK)]

        for c in range(NBUF):
            gather(c, c)
        for c in range(NJOBS):
            b = c % NBUF
            pltpu.make_async_copy(tabs[c // N_CHUNKS].at[idx_v.at[c]], rows[b], gsems[b]).wait()
            pltpu.async_copy(rows[b], out_slice(c), osems[b])
            nc = c + NBUF
            if nc < NJOBS:
                pltpu.make_async_copy(rows[b], out_slice(c), osems[b]).wait()
                gather(nc, b)
        for c in range(NJOBS - NBUF, NJOBS):
            b = c % NBUF
            pltpu.make_async_copy(rows[b], out_slice(c), osems[b]).wait()

    return k(head2, rel2, tail2, ent, rtab)


@jax.jit
def kernel(head, rel, tail, entity_embedding, rel_embedding):
    head2 = head.astype(jnp.int32).reshape(BATCH // CHUNK, CHUNK)
    rel2 = rel.astype(jnp.int32).reshape(BATCH // CHUNK, CHUNK)
    tail2 = tail.astype(jnp.int32).reshape(BATCH // CHUNK, CHUNK)
    return _sc_gather3(head2, rel2, tail2, entity_embedding, rel_embedding)


# speedup vs baseline: 3.0587x; 1.1642x over previous
import functools

import jax
import jax.numpy as jnp
from jax import lax
from jax.experimental import pallas as pl
from jax.experimental.pallas import tpu as pltpu
from jax.experimental.pallas import tpu_sc as plsc

NC = 2
NS = 16
NW = NC * NS

BATCH = 16384
DIM = 128
NUM_RELS = 1000
B_PER_W = BATCH // NW
CHUNK = 128
N_CHUNKS = B_PER_W // CHUNK


def _sc_gather3(head2, rel2, tail2, ent, rtab):
    mesh = plsc.VectorSubcoreMesh(
        core_axis_name="c", subcore_axis_name="s", num_cores=NC, num_subcores=NS
    )
    out_t = (
        jax.ShapeDtypeStruct((BATCH, DIM), jnp.float32),
        jax.ShapeDtypeStruct((BATCH, DIM), jnp.float32),
        jax.ShapeDtypeStruct((BATCH, DIM), jnp.float32),
    )

    NBUF = 7
    NJOBS = 3 * N_CHUNKS
    ORDER = []
    for cc in range(N_CHUNKS):
        ORDER += [(0, cc), (2, cc), (1, cc)]

    @functools.partial(
        pl.kernel,
        out_type=out_t,
        mesh=mesh,
        scratch_types=[
            pltpu.VMEM((NJOBS, CHUNK), jnp.int32),
            [pltpu.VMEM((CHUNK, DIM), jnp.float32) for _ in range(NBUF)],
            pltpu.VMEM_SHARED((NUM_RELS, DIM), jnp.float32),
            [pltpu.SemaphoreType.DMA for _ in range(NBUF)],
            [pltpu.SemaphoreType.DMA for _ in range(NBUF)],
            pltpu.SemaphoreType.DMA,
            pltpu.SemaphoreType.DMA,
        ],
    )
    def k(head_h, rel_h, tail_h, ent_h, rtab_h, ho, ro, to,
          idx_v, rows, rtab_s, gsems, osems, isem, tsem):
        sid = lax.axis_index("s")
        wid = sid * NC + lax.axis_index("c")
        rbase = wid * N_CHUNKS
        obase = wid * B_PER_W

        outs = (ho, ro, to)

        @pl.when(sid == 0)
        def _():
            pltpu.async_copy(rtab_h, rtab_s, tsem)

        for j, idx_h in enumerate((head_h, rel_h, tail_h)):
            pltpu.async_copy(idx_h.at[pl.ds(rbase, N_CHUNKS)],
                             idx_v.at[pl.ds(j * N_CHUNKS, N_CHUNKS)], isem)
        for j, idx_h in enumerate((head_h, rel_h, tail_h)):
            pltpu.make_async_copy(idx_h.at[pl.ds(rbase, N_CHUNKS)],
                                  idx_v.at[pl.ds(j * N_CHUNKS, N_CHUNKS)], isem).wait()

        @pl.when(sid == 0)
        def _():
            pltpu.make_async_copy(rtab_h, rtab_s, tsem).wait()

        plsc.subcore_barrier()

        def gather(i, b):
            j, cc = ORDER[i]
            tab = rtab_s if j == 1 else ent_h
            pltpu.async_copy(tab.at[idx_v.at[j * N_CHUNKS + cc]], rows[b], gsems[b])

        def out_copy(i, b):
            j, cc = ORDER[i]
            return pltpu.make_async_copy(
                rows[b], outs[j].at[pl.ds(obase + cc * CHUNK, CHUNK)], osems[b])

        for i in range(NBUF):
            gather(i, i)
        for i in range(NJOBS):
            b = i % NBUF
            j, cc = ORDER[i]
            tab = rtab_s if j == 1 else ent_h
            pltpu.make_async_copy(tab.at[idx_v.at[j * N_CHUNKS + cc]], rows[b], gsems[b]).wait()
            out_copy(i, b).start()
            ni = i + NBUF
            if ni < NJOBS:
                out_copy(i, b).wait()
                gather(ni, b)
        for i in range(NJOBS - NBUF, NJOBS):
            out_copy(i, i % NBUF).wait()

    return k(head2, rel2, tail2, ent, rtab)


@jax.jit
def kernel(head, rel, tail, entity_embedding, rel_embedding):
    head2 = head.astype(jnp.int32).reshape(BATCH // CHUNK, CHUNK)
    rel2 = rel.astype(jnp.int32).reshape(BATCH // CHUNK, CHUNK)
    tail2 = tail.astype(jnp.int32).reshape(BATCH // CHUNK, CHUNK)
    return _sc_gather3(head2, rel2, tail2, entity_embedding, rel_embedding)
